# baseline (speedup 1.0000x reference)
import functools

import jax
import jax.numpy as jnp
from jax import lax
from jax.experimental import pallas as pl
from jax.experimental.pallas import tpu as pltpu
from jax.experimental.pallas import tpu_sc as plsc

D = 64
W = 2 * D
SCALE = 8.0
L = 16

NC, NS = 2, 16
NW = NC * NS

R, S = 4096, 200
RB = R // NW
NPAIR = S // 2

_mesh = plsc.VectorSubcoreMesh(core_axis_name="c", subcore_axis_name="s")


@functools.partial(
    pl.kernel,
    out_type=jax.ShapeDtypeStruct((S, D // 8, R // RB, 8, RB), jnp.float32),
    mesh=_mesh,
    scratch_types=[
        pltpu.VMEM((S, RB), jnp.int32),
        pltpu.VMEM((RB,), jnp.int32),
        pltpu.VMEM((RB,), jnp.int32),
        pltpu.VMEM((RB, W), jnp.float32),
        pltpu.VMEM((RB, W), jnp.float32),
        pltpu.VMEM((D // 8, 1, 8, RB), jnp.float32),
        pltpu.VMEM((D // 8, 1, 8, RB), jnp.float32),
        pltpu.SemaphoreType.DMA,
        pltpu.SemaphoreType.DMA,
        pltpu.SemaphoreType.DMA,
        pltpu.SemaphoreType.DMA,
    ],
    compiler_params=pltpu.CompilerParams(
        use_tc_tiling_on_sc=False, needs_layout_passes=False),
)
def _embed_sc(xt_hbm, tab_hbm, out5_hbm, xbuf, ib0, ib1, rows0, rows1,
              tb0, tb1, gsem0, gsem1, ssem0, ssem1):
    wid = lax.axis_index("s") * NC + lax.axis_index("c")
    bias = jnp.full((L,), SCALE, jnp.float32)
    rvec = [lax.iota(jnp.int32, L) + rc * L for rc in range(RB // L)]

    ib_v = (ib0, ib1)
    rows_v = (rows0, rows1)
    tb_v = (tb0, tb1)
    gsem = (gsem0, gsem1)
    ssem = (ssem0, ssem1)

    def prep_idx(s, b):
        for rc in range(RB // L):
            sl = pl.ds(rc * L, L)
            ib_v[b][sl] = jax.lax.shift_right_logical(xbuf[s, sl], 1)

    def fire_gather(b):
        pltpu.async_copy(tab_hbm.at[ib_v[b]], rows_v[b], gsem[b])

    def wait_gather(b):
        pltpu.make_async_copy(
            tab_hbm.at[ib_v[b]], rows_v[b], gsem[b]).wait()

    def fire_store(s, b):
        pltpu.async_copy(
            tb_v[b], out5_hbm.at[s, :, pl.ds(wid, 1)], ssem[b])

    def wait_store(s, b):
        pltpu.make_async_copy(
            tb_v[b], out5_hbm.at[s, :, pl.ds(wid, 1)], ssem[b]).wait()

    def transpose_bias(s, b):
        one = jnp.full((L,), 1, jnp.int32)
        cols = [
            jax.lax.shift_left(xbuf[s, pl.ds(rc * L, L)] & one, 6)
            for rc in range(RB // L)
        ]

        @plsc.parallel_loop(0, D, unroll=8)
        def _(d):
            for rc in range(RB // L):
                v = plsc.load_gather(rows_v[b], [rvec[rc], cols[rc] + d])
                tb_v[b][d // 8, 0, d % 8, pl.ds(rc * L, L)] = v + bias

    pltpu.sync_copy(xt_hbm.at[:, pl.ds(wid * RB, RB)], xbuf)
    prep_idx(0, 0)
    fire_gather(0)

    def pair_body(p, carry):
        sa = 2 * p
        sb = sa + 1

        prep_idx(sb, 1)
        fire_gather(1)
        wait_gather(0)

        @pl.when(p > 0)
        def _():
            wait_store(sa - 2, 0)
        transpose_bias(sa, 0)
        fire_store(sa, 0)

        @pl.when(p < NPAIR - 1)
        def _():
            prep_idx(sb + 1, 0)
            fire_gather(0)
        wait_gather(1)

        @pl.when(p > 0)
        def _():
            wait_store(sb - 2, 1)
        transpose_bias(sb, 1)
        fire_store(sb, 1)
        return carry

    lax.fori_loop(0, NPAIR, pair_body, 0)

    wait_store(S - 2, 0)
    wait_store(S - 1, 1)


def kernel(x, embedding_table):
    out5 = _embed_sc(x.T, embedding_table.reshape(500000, W))
    return out5.transpose(2, 4, 0, 1, 3).reshape(R, S, D)


# speedup vs baseline: 1.0678x; 1.0678x over previous
import functools

import jax
import jax.numpy as jnp
from jax import lax
from jax.experimental import pallas as pl
from jax.experimental.pallas import tpu as pltpu
from jax.experimental.pallas import tpu_sc as plsc

D = 64
W = 2 * D
SCALE = 8.0
L = 16

NC, NS = 2, 16
NW = NC * NS

R, S = 4096, 200
RB = R // NW
NPAIR = S // 2

_mesh = plsc.VectorSubcoreMesh(core_axis_name="c", subcore_axis_name="s")


@functools.partial(
    pl.kernel,
    out_type=jax.ShapeDtypeStruct((S, D // 8, R // RB, 8, RB), jnp.float32),
    mesh=_mesh,
    scratch_types=[
        pltpu.VMEM((S, RB), jnp.int32),
        pltpu.VMEM((RB,), jnp.int32),
        pltpu.VMEM((RB,), jnp.int32),
        pltpu.VMEM((RB, W), jnp.float32),
        pltpu.VMEM((RB, W), jnp.float32),
        pltpu.VMEM((D // 8, 1, 8, RB), jnp.float32),
        pltpu.VMEM((D // 8, 1, 8, RB), jnp.float32),
        pltpu.SemaphoreType.DMA,
        pltpu.SemaphoreType.DMA,
        pltpu.SemaphoreType.DMA,
        pltpu.SemaphoreType.DMA,
    ],
    compiler_params=pltpu.CompilerParams(
        use_tc_tiling_on_sc=True, needs_layout_passes=False),
)
def _embed_sc(xt_hbm, tab_hbm, out5_hbm, xbuf, ib0, ib1, rows0, rows1,
              tb0, tb1, gsem0, gsem1, ssem0, ssem1):
    wid = lax.axis_index("s") * NC + lax.axis_index("c")
    bias = jnp.full((L,), SCALE, jnp.float32)
    rvec = [lax.iota(jnp.int32, L) + rc * L for rc in range(RB // L)]

    ib_v = (ib0, ib1)
    rows_v = (rows0, rows1)
    tb_v = (tb0, tb1)
    gsem = (gsem0, gsem1)
    ssem = (ssem0, ssem1)

    def prep_idx(s, b):
        for rc in range(RB // L):
            sl = pl.ds(rc * L, L)
            ib_v[b][sl] = jax.lax.shift_right_logical(xbuf[s, sl], 1)

    def fire_gather(b):
        pltpu.async_copy(tab_hbm.at[ib_v[b]], rows_v[b], gsem[b])

    def wait_gather(b):
        pltpu.make_async_copy(
            tab_hbm.at[ib_v[b]], rows_v[b], gsem[b]).wait()

    def fire_store(s, b):
        pltpu.async_copy(
            tb_v[b], out5_hbm.at[s, :, pl.ds(wid, 1)], ssem[b])

    def wait_store(s, b):
        pltpu.make_async_copy(
            tb_v[b], out5_hbm.at[s, :, pl.ds(wid, 1)], ssem[b]).wait()

    def transpose_bias(s, b):
        one = jnp.full((L,), 1, jnp.int32)
        for half in range(2):
            rcs = range(4 * half, 4 * half + 4)
            cols = {
                rc: jax.lax.shift_left(xbuf[s, pl.ds(rc * L, L)] & one, 6)
                for rc in rcs
            }

            @plsc.parallel_loop(0, D, unroll=2)
            def _(d):
                for rc in rcs:
                    v = plsc.load_gather(
                        rows_v[b], [rvec[rc], cols[rc] + d])
                    tb_v[b][d // 8, 0, d % 8, pl.ds(rc * L, L)] = v + bias

    pltpu.sync_copy(xt_hbm.at[:, pl.ds(wid * RB, RB)], xbuf)
    prep_idx(0, 0)
    fire_gather(0)

    def pair_body(p, carry):
        sa = 2 * p
        sb = sa + 1

        prep_idx(sb, 1)
        fire_gather(1)
        wait_gather(0)

        @pl.when(p > 0)
        def _():
            wait_store(sa - 2, 0)
        transpose_bias(sa, 0)
        fire_store(sa, 0)

        @pl.when(p < NPAIR - 1)
        def _():
            prep_idx(sb + 1, 0)
            fire_gather(0)
        wait_gather(1)

        @pl.when(p > 0)
        def _():
            wait_store(sb - 2, 1)
        transpose_bias(sb, 1)
        fire_store(sb, 1)
        return carry

    lax.fori_loop(0, NPAIR, pair_body, 0)

    wait_store(S - 2, 0)
    wait_store(S - 1, 1)


def kernel(x, embedding_table):
    out5 = _embed_sc(x.T, embedding_table.reshape(500000, W))
    return out5.transpose(2, 4, 0, 1, 3).reshape(R, S, D)


# speedup vs baseline: 1.0939x; 1.0244x over previous
import functools

import jax
import jax.numpy as jnp
from jax import lax
from jax.experimental import pallas as pl
from jax.experimental.pallas import tpu as pltpu
from jax.experimental.pallas import tpu_sc as plsc

D = 64
SCALE = 8.0
L = 16

NC, NS = 2, 16
NW = NC * NS

B = 4096 * 200
BPW = B // NW
CH = 512
SUB = 128
NSUB = CH // SUB
NCHUNK = BPW // CH
NPAIR = NCHUNK // 2

_mesh = plsc.VectorSubcoreMesh(core_axis_name="c", subcore_axis_name="s")


@functools.partial(
    pl.kernel,
    out_type=jax.ShapeDtypeStruct((B, D), jnp.float32),
    mesh=_mesh,
    scratch_types=[
        pltpu.VMEM((CH,), jnp.int32),
        pltpu.VMEM((CH,), jnp.int32),
        pltpu.VMEM((CH, D), jnp.float32),
        pltpu.VMEM((CH, D), jnp.float32),
        pltpu.SemaphoreType.DMA,
        pltpu.SemaphoreType.DMA,
        pltpu.SemaphoreType.DMA,
        pltpu.SemaphoreType.DMA,
        pltpu.SemaphoreType.DMA,
        pltpu.SemaphoreType.DMA,
    ],
    compiler_params=pltpu.CompilerParams(use_tc_tiling_on_sc=False),
)
def _embed_sc(x_hbm, tab_hbm, out_hbm, idx0, idx1, rows0, rows1,
              isem0, isem1, gsem0, gsem1, ssem0, ssem1):
    wid = lax.axis_index("s") * NC + lax.axis_index("c")
    base = wid * BPW
    bias = jnp.full((L,), SCALE, jnp.float32)

    idx_v = (idx0, idx1)
    rows_v = (rows0, rows1)
    isem = (isem0, isem1)
    gsem = (gsem0, gsem1)
    ssem = (ssem0, ssem1)

    def fire_idx(g, b):
        pltpu.async_copy(x_hbm.at[pl.ds(base + g * CH, CH)], idx_v[b], isem[b])

    def wait_idx(g, b):
        pltpu.make_async_copy(
            x_hbm.at[pl.ds(base + g * CH, CH)], idx_v[b], isem[b]).wait()

    def fire_gathers(b):
        for j in range(NSUB):
            pltpu.async_copy(
                tab_hbm.at[idx_v[b].at[pl.ds(j * SUB, SUB)]],
                rows_v[b].at[pl.ds(j * SUB, SUB)],
                gsem[b],
            )

    def wait_gathers(b):
        for j in range(NSUB):
            pltpu.make_async_copy(
                tab_hbm.at[idx_v[b].at[pl.ds(j * SUB, SUB)]],
                rows_v[b].at[pl.ds(j * SUB, SUB)],
                gsem[b],
            ).wait()

    def fire_store(g, b):
        pltpu.async_copy(rows_v[b], out_hbm.at[pl.ds(base + g * CH, CH)], ssem[b])

    def wait_store(g, b):
        pltpu.make_async_copy(
            rows_v[b], out_hbm.at[pl.ds(base + g * CH, CH)], ssem[b]).wait()

    def add_bias(b):
        @plsc.parallel_loop(0, CH, unroll=4)
        def _(i):
            for k in range(D // L):
                sl = pl.ds(k * L, L)
                rows_v[b][i, sl] = rows_v[b][i, sl] + bias

    fire_idx(0, 0)
    fire_idx(1, 1)
    wait_idx(0, 0)
    fire_gathers(0)

    def pair_body(p, carry):
        ga = 2 * p
        gb = ga + 1

        wait_idx(gb, 1)

        @pl.when(p > 0)
        def _():
            wait_store(ga - 1, 1)
        fire_gathers(1)

        wait_gathers(0)

        @pl.when(p < NPAIR - 1)
        def _():
            fire_idx(ga + 2, 0)
        add_bias(0)
        fire_store(ga, 0)

        @pl.when(p < NPAIR - 1)
        def _():
            wait_idx(gb + 1, 0)
            wait_store(ga, 0)
            fire_gathers(0)

        wait_gathers(1)

        @pl.when(p < NPAIR - 1)
        def _():
            fire_idx(gb + 2, 1)
        add_bias(1)
        fire_store(gb, 1)
        return carry

    lax.fori_loop(0, NPAIR, pair_body, 0)

    wait_store(NCHUNK - 2, 0)
    wait_store(NCHUNK - 1, 1)


def kernel(x, embedding_table):
    out = _embed_sc(x.reshape(B), embedding_table)
    return out.reshape(x.shape + (D,))
